# baseline (speedup 1.0000x reference)
import functools

import jax
import jax.numpy as jnp
import numpy as np
from jax import lax
from jax.experimental import pallas as pl
from jax.experimental.pallas import tpu as pltpu
from jax.experimental.pallas import tpu_sc as plsc

N = 10000
E = 320000
D = 128
H = 8
DH = 16

NC = 2
NS = 16
NW = NC * NS
C = 48
NCHUNK = 210
EPT = C * NCHUNK
E_PAD = NW * EPT
G = C // 16
NP = 10112
RPT = NP // NS


def _qkv_body(h_ref, wq_ref, wk_ref, wv_ref, q_out, k_out, v_out):
    x = h_ref[...]
    dn = (((1,), (1,)), ((), ()))
    q_out[...] = lax.dot_general(x, wq_ref[...], dn,
                                 preferred_element_type=jnp.float32)
    k_out[...] = lax.dot_general(x, wk_ref[...], dn,
                                 preferred_element_type=jnp.float32)
    v_out[...] = lax.dot_general(x, wv_ref[...], dn,
                                 preferred_element_type=jnp.float32)


def _edge_body(q_hbm, k_hbm, v_hbm, ei_hbm,
               wv_out, z_out,
               k0, q0, k1, q1, v_buf, i0, i1, z_o, z_s,
               wv_acc, z_acc,
               semk0, semq0, semk1, semq1, semv, semw, semz):
    cid = lax.axis_index("c")
    sid = lax.axis_index("s")
    wid = sid * NC + cid
    lv = lax.iota(jnp.int32, 16)

    def zrow(r, _):
        for hh in range(8):
            v_buf[r, pl.ds(hh * 16, 16)] = jnp.zeros((16,), jnp.float32)
        return 0
    lax.fori_loop(0, C, zrow, 0)
    zro = jnp.zeros((16,), jnp.float32)
    for t in range(C // 2):
        rows = 2 * t + (lv >> 3)
        cols = lv & 7
        plsc.store_scatter(z_o, [rows, cols], zro)
    for j in range(RPT // C):
        base = sid * RPT + j * C
        pltpu.sync_copy(v_buf, wv_acc.at[pl.ds(base, C)])
        pltpu.sync_copy(z_o, z_acc.at[pl.ds(base, C)])
    rem = RPT - (RPT // C) * C
    base = sid * RPT + (RPT // C) * C
    pltpu.sync_copy(v_buf.at[pl.ds(0, rem)], wv_acc.at[pl.ds(base, rem)])
    pltpu.sync_copy(z_o.at[pl.ds(0, rem)], z_acc.at[pl.ds(base, rem)])
    plsc.subcore_barrier()

    def issue(ci, kb, qb, idx, semk, semq):
        base = wid * EPT + ci * C
        pltpu.sync_copy(ei_hbm.at[:, pl.ds(base, C)], idx)
        pltpu.async_copy(k_hbm.at[idx.at[0]], kb, semk)
        pltpu.async_copy(q_hbm.at[idx.at[1]], qb, semq)

    def process(ci, kb, qb, idx, semk, semq):
        src_i = idx.at[0]
        dst_i = idx.at[1]
        cv = pltpu.async_copy(v_hbm.at[src_i], v_buf, semv)
        pltpu.make_async_copy(k_hbm.at[src_i], kb, semk).wait()
        pltpu.make_async_copy(q_hbm.at[dst_i], qb, semq).wait()

        def score_body(g, _):
            ev = g * 16 + lv
            for h in range(H):
                acc0 = jnp.zeros((16,), jnp.float32)
                acc1 = jnp.zeros((16,), jnp.float32)
                for i in range(DH):
                    cvec = h * 16 + ((lv + i) & 15)
                    kv = plsc.load_gather(kb, [ev, cvec])
                    qv = plsc.load_gather(qb, [ev, cvec])
                    if i % 2 == 0:
                        acc0 = acc0 + kv * qv
                    else:
                        acc1 = acc1 + kv * qv
                sh = jnp.exp(jnp.clip((acc0 + acc1) * 0.25, -5.0, 5.0))
                plsc.store_scatter(z_s, [ev, jnp.full((16,), h, jnp.int32)], sh)
            return 0
        lax.fori_loop(0, G, score_body, 0)

        def zcopy_body(t, _):
            rows = 2 * t + (lv >> 3)
            cols = lv & 7
            val = plsc.load_gather(z_s, [rows, cols])
            plsc.store_scatter(z_o, [rows, cols], val)
            return 0
        lax.fori_loop(0, C // 2, zcopy_body, 0)

        cv.wait()

        def wv_body(g, _):
            ev = g * 16 + lv
            for h in range(H):
                sh = plsc.load_gather(z_s, [ev, jnp.full((16,), h, jnp.int32)])
                for i in range(DH):
                    cvec = h * 16 + ((lv + i) & 15)
                    vv = plsc.load_gather(v_buf, [ev, cvec])
                    plsc.store_scatter(v_buf, [ev, cvec], vv * sh)
            return 0
        lax.fori_loop(0, G, wv_body, 0)

        sa = pltpu.async_copy(v_buf, wv_acc.at[dst_i], semw, add=True)
        sz = pltpu.async_copy(z_o, z_acc.at[dst_i], semz, add=True)
        sa.wait()
        sz.wait()

    issue(0, k0, q0, i0, semk0, semq0)

    def pair_body(p, _):
        c0 = 2 * p
        issue(c0 + 1, k1, q1, i1, semk1, semq1)
        process(c0, k0, q0, i0, semk0, semq0)
        issue(c0 + 2, k0, q0, i0, semk0, semq0)
        process(c0 + 1, k1, q1, i1, semk1, semq1)
        return 0
    lax.fori_loop(0, NCHUNK // 2 - 1, pair_body, 0)

    issue(NCHUNK - 1, k1, q1, i1, semk1, semq1)
    process(NCHUNK - 2, k0, q0, i0, semk0, semq0)
    process(NCHUNK - 1, k1, q1, i1, semk1, semq1)

    plsc.subcore_barrier()

    for j in range(RPT // C):
        base = sid * RPT + j * C
        pltpu.sync_copy(wv_acc.at[pl.ds(base, C)],
                        wv_out.at[cid, pl.ds(base, C)])
        pltpu.sync_copy(z_acc.at[pl.ds(base, C)],
                        z_out.at[cid, pl.ds(base, C)])
    base = sid * RPT + (RPT // C) * C
    pltpu.sync_copy(wv_acc.at[pl.ds(base, rem)],
                    wv_out.at[cid, pl.ds(base, rem)])
    pltpu.sync_copy(z_acc.at[pl.ds(base, rem)],
                    z_out.at[cid, pl.ds(base, rem)])


_edge_kernel = functools.partial(
    pl.kernel,
    out_type=[jax.ShapeDtypeStruct((NC, NP, D), jnp.float32),
              jax.ShapeDtypeStruct((NC, NP, 8), jnp.float32)],
    mesh=plsc.VectorSubcoreMesh(core_axis_name="c", subcore_axis_name="s"),
    compiler_params=pltpu.CompilerParams(needs_layout_passes=False,
                                         use_tc_tiling_on_sc=False),
    scratch_types=[
        pltpu.VMEM((C, D), jnp.float32),
        pltpu.VMEM((C, D), jnp.float32),
        pltpu.VMEM((C, D), jnp.float32),
        pltpu.VMEM((C, D), jnp.float32),
        pltpu.VMEM((C, D), jnp.float32),
        pltpu.VMEM((2, C), jnp.int32),
        pltpu.VMEM((2, C), jnp.int32),
        pltpu.VMEM((C, 8), jnp.float32),
        pltpu.VMEM((C, 17), jnp.float32),
        pltpu.VMEM_SHARED((NP, D), jnp.float32),
        pltpu.VMEM_SHARED((NP, 8), jnp.float32),
        pltpu.SemaphoreType.DMA,
        pltpu.SemaphoreType.DMA,
        pltpu.SemaphoreType.DMA,
        pltpu.SemaphoreType.DMA,
        pltpu.SemaphoreType.DMA,
        pltpu.SemaphoreType.DMA,
        pltpu.SemaphoreType.DMA,
    ],
)(_edge_body)


def _post_body(h_ref, wvp_ref, zp_ref, s_ref, wo_ref, bo_ref,
               w1_ref, b1_ref, w2_ref, b2_ref,
               g1_ref, be1_ref, g2_ref, be2_ref, out_ref):
    wv = wvp_ref[0, 0:N] + wvp_ref[1, 0:N]
    z = zp_ref[0, 0:N] + zp_ref[1, 0:N]
    dn = (((1,), (1,)), ((), ()))
    dn0 = (((1,), (0,)), ((), ()))
    zx = lax.dot_general(1.0 / z, s_ref[...], dn0,
                         preferred_element_type=jnp.float32)
    head = wv * zx
    hh = lax.dot_general(head, wo_ref[...], dn,
                         preferred_element_type=jnp.float32) + bo_ref[...]
    hh = h_ref[...] + hh
    mu = jnp.mean(hh, axis=0)
    var = jnp.mean((hh - mu) ** 2, axis=0)
    hh = (hh - mu) * lax.rsqrt(var + 1e-5) * g1_ref[...] + be1_ref[...]
    f = lax.dot_general(hh, w1_ref[...], dn,
                        preferred_element_type=jnp.float32) + b1_ref[...]
    f = jnp.maximum(f, 0.0)
    f = lax.dot_general(f, w2_ref[...], dn,
                        preferred_element_type=jnp.float32) + b2_ref[...]
    hh = hh + f
    mu2 = jnp.mean(hh, axis=0)
    var2 = jnp.mean((hh - mu2) ** 2, axis=0)
    out_ref[...] = ((hh - mu2) * lax.rsqrt(var2 + 1e-5) * g2_ref[...]
                    + be2_ref[...])


_S = np.repeat(np.eye(H, dtype=np.float32), DH, axis=1)


def kernel(h, edge_index, pos_enc, WQ, WK, WV, WO, bO, W1, b1, W2, b2,
           g1, be1, g2, be2):
    ei = edge_index.astype(jnp.int32)
    pad = jnp.concatenate(
        [jnp.zeros((1, E_PAD - E), jnp.int32),
         jnp.full((1, E_PAD - E), NP - 1, jnp.int32)], axis=0)
    ei = jnp.concatenate([ei, pad], axis=1)

    q, k, v = pl.pallas_call(
        _qkv_body,
        out_shape=[jax.ShapeDtypeStruct((N, D), jnp.float32)] * 3,
    )(h, WQ, WK, WV)
    q = jnp.concatenate([q, jnp.zeros((NP - N, D), jnp.float32)], axis=0)

    wvp, zp = _edge_kernel(q, k, v, ei)

    out = pl.pallas_call(
        _post_body,
        out_shape=jax.ShapeDtypeStruct((N, D), jnp.float32),
    )(h, wvp, zp, jnp.asarray(_S), WO, bO, W1, b1, W2, b2, g1, be1, g2, be2)
    return out


# speedup vs baseline: 1.1486x; 1.1486x over previous
import functools

import jax
import jax.numpy as jnp
import numpy as np
from jax import lax
from jax.experimental import pallas as pl
from jax.experimental.pallas import tpu as pltpu
from jax.experimental.pallas import tpu_sc as plsc

N = 10000
E = 320000
D = 128
H = 8
DH = 16

NC = 2
NS = 16
NW = NC * NS
C = 80
NCHUNK = 126
EPT = C * NCHUNK
E_PAD = NW * EPT
G = C // 16
NP = 10112
RPT = NP // NS


def _qkv_body(h_ref, wq_ref, wk_ref, wv_ref, q_out, k_out, v_out):
    x = h_ref[...]
    dn = (((1,), (1,)), ((), ()))
    q_out[...] = lax.dot_general(x, wq_ref[...], dn,
                                 preferred_element_type=jnp.float32
                                 ).astype(jnp.bfloat16)
    k_out[...] = lax.dot_general(x, wk_ref[...], dn,
                                 preferred_element_type=jnp.float32
                                 ).astype(jnp.bfloat16)
    v_out[...] = lax.dot_general(x, wv_ref[...], dn,
                                 preferred_element_type=jnp.float32)


def _edge_body(q_hbm, k_hbm, v_hbm, ei_hbm,
               wv_out, z_out,
               k0, q0, k1, q1, v_buf, i0, i1, z_o, z_s,
               wv_acc, z_acc,
               semk0, semq0, semk1, semq1, semv, semw, semz):
    cid = lax.axis_index("c")
    sid = lax.axis_index("s")
    wid = sid * NC + cid
    lv = lax.iota(jnp.int32, 16)

    def zrow(r, _):
        for hh in range(8):
            v_buf[r, pl.ds(hh * 16, 16)] = jnp.zeros((16,), jnp.float32)
        return 0
    lax.fori_loop(0, C, zrow, 0)
    zro = jnp.zeros((16,), jnp.float32)
    for t in range(C // 2):
        rows = 2 * t + (lv >> 3)
        cols = lv & 7
        plsc.store_scatter(z_o, [rows, cols], zro)
    for j in range(RPT // C):
        base = sid * RPT + j * C
        pltpu.sync_copy(v_buf, wv_acc.at[pl.ds(base, C)])
        pltpu.sync_copy(z_o, z_acc.at[pl.ds(base, C)])
    rem = RPT - (RPT // C) * C
    base = sid * RPT + (RPT // C) * C
    pltpu.sync_copy(v_buf.at[pl.ds(0, rem)], wv_acc.at[pl.ds(base, rem)])
    pltpu.sync_copy(z_o.at[pl.ds(0, rem)], z_acc.at[pl.ds(base, rem)])
    plsc.subcore_barrier()

    def issue(ci, kb, qb, idx, semk, semq):
        base = wid * EPT + ci * C
        pltpu.sync_copy(ei_hbm.at[:, pl.ds(base, C)], idx)
        pltpu.async_copy(k_hbm.at[idx.at[0]], kb, semk)
        pltpu.async_copy(q_hbm.at[idx.at[1]], qb, semq)

    def process(ci, kb, qb, idx, semk, semq):
        src_i = idx.at[0]
        dst_i = idx.at[1]
        cv = pltpu.async_copy(v_hbm.at[src_i], v_buf, semv)
        pltpu.make_async_copy(k_hbm.at[src_i], kb, semk).wait()
        pltpu.make_async_copy(q_hbm.at[dst_i], qb, semq).wait()

        def score_body(g, _):
            ev = g * 16 + lv
            for h in range(H):
                acc = [jnp.zeros((16,), jnp.float32) for _ in range(4)]
                for i in range(DH // 2):
                    cvec = h * 8 + ((lv + i) & 7)
                    kw = plsc.load_gather(kb, [ev, cvec])
                    qw = plsc.load_gather(qb, [ev, cvec])
                    ka, kx = plsc.unpack(plsc.bitcast(kw, jnp.bfloat16),
                                         format=plsc.PackFormat.INTERLEAVED)
                    qa, qx = plsc.unpack(plsc.bitcast(qw, jnp.bfloat16),
                                         format=plsc.PackFormat.INTERLEAVED)
                    acc[2 * (i % 2)] = acc[2 * (i % 2)] + ka * qa
                    acc[2 * (i % 2) + 1] = acc[2 * (i % 2) + 1] + kx * qx
                sh = jnp.exp(jnp.clip(
                    ((acc[0] + acc[1]) + (acc[2] + acc[3])) * 0.25, -5.0, 5.0))
                plsc.store_scatter(z_s, [ev, jnp.full((16,), h, jnp.int32)], sh)
            return 0
        lax.fori_loop(0, G, score_body, 0)

        def zcopy_body(t, _):
            rows = 2 * t + (lv >> 3)
            cols = lv & 7
            val = plsc.load_gather(z_s, [rows, cols])
            plsc.store_scatter(z_o, [rows, cols], val)
            return 0
        lax.fori_loop(0, C // 2, zcopy_body, 0)

        cv.wait()

        def wv_body(g, _):
            ev = g * 16 + lv
            for h in range(H):
                sh = plsc.load_gather(z_s, [ev, jnp.full((16,), h, jnp.int32)])
                for i in range(DH):
                    cvec = h * 16 + ((lv + i) & 15)
                    vv = plsc.load_gather(v_buf, [ev, cvec])
                    plsc.store_scatter(v_buf, [ev, cvec], vv * sh)
            return 0
        lax.fori_loop(0, G, wv_body, 0)

        sa = pltpu.async_copy(v_buf, wv_acc.at[dst_i], semw, add=True)
        sz = pltpu.async_copy(z_o, z_acc.at[dst_i], semz, add=True)
        sa.wait()
        sz.wait()

    issue(0, k0, q0, i0, semk0, semq0)

    def pair_body(p, _):
        c0 = 2 * p
        issue(c0 + 1, k1, q1, i1, semk1, semq1)
        process(c0, k0, q0, i0, semk0, semq0)
        issue(c0 + 2, k0, q0, i0, semk0, semq0)
        process(c0 + 1, k1, q1, i1, semk1, semq1)
        return 0
    lax.fori_loop(0, NCHUNK // 2 - 1, pair_body, 0)

    issue(NCHUNK - 1, k1, q1, i1, semk1, semq1)
    process(NCHUNK - 2, k0, q0, i0, semk0, semq0)
    process(NCHUNK - 1, k1, q1, i1, semk1, semq1)

    plsc.subcore_barrier()

    for j in range(RPT // C):
        base = sid * RPT + j * C
        pltpu.sync_copy(wv_acc.at[pl.ds(base, C)],
                        wv_out.at[cid, pl.ds(base, C)])
        pltpu.sync_copy(z_acc.at[pl.ds(base, C)],
                        z_out.at[cid, pl.ds(base, C)])
    base = sid * RPT + (RPT // C) * C
    pltpu.sync_copy(wv_acc.at[pl.ds(base, rem)],
                    wv_out.at[cid, pl.ds(base, rem)])
    pltpu.sync_copy(z_acc.at[pl.ds(base, rem)],
                    z_out.at[cid, pl.ds(base, rem)])


_edge_kernel = functools.partial(
    pl.kernel,
    out_type=[jax.ShapeDtypeStruct((NC, NP, D), jnp.float32),
              jax.ShapeDtypeStruct((NC, NP, 8), jnp.float32)],
    mesh=plsc.VectorSubcoreMesh(core_axis_name="c", subcore_axis_name="s"),
    compiler_params=pltpu.CompilerParams(needs_layout_passes=False,
                                         use_tc_tiling_on_sc=False),
    scratch_types=[
        pltpu.VMEM((C, D // 2), jnp.int32),
        pltpu.VMEM((C, D // 2), jnp.int32),
        pltpu.VMEM((C, D // 2), jnp.int32),
        pltpu.VMEM((C, D // 2), jnp.int32),
        pltpu.VMEM((C, D), jnp.float32),
        pltpu.VMEM((2, C), jnp.int32),
        pltpu.VMEM((2, C), jnp.int32),
        pltpu.VMEM((C, 8), jnp.float32),
        pltpu.VMEM((C, 17), jnp.float32),
        pltpu.VMEM_SHARED((NP, D), jnp.float32),
        pltpu.VMEM_SHARED((NP, 8), jnp.float32),
        pltpu.SemaphoreType.DMA,
        pltpu.SemaphoreType.DMA,
        pltpu.SemaphoreType.DMA,
        pltpu.SemaphoreType.DMA,
        pltpu.SemaphoreType.DMA,
        pltpu.SemaphoreType.DMA,
        pltpu.SemaphoreType.DMA,
    ],
)(_edge_body)


def _post_body(h_ref, wvp_ref, zp_ref, s_ref, wo_ref, bo_ref,
               w1_ref, b1_ref, w2_ref, b2_ref,
               g1_ref, be1_ref, g2_ref, be2_ref, out_ref):
    wv = wvp_ref[0, 0:N] + wvp_ref[1, 0:N]
    z = zp_ref[0, 0:N] + zp_ref[1, 0:N]
    dn = (((1,), (1,)), ((), ()))
    dn0 = (((1,), (0,)), ((), ()))
    zx = lax.dot_general(1.0 / z, s_ref[...], dn0,
                         preferred_element_type=jnp.float32)
    head = wv * zx
    hh = lax.dot_general(head, wo_ref[...], dn,
                         preferred_element_type=jnp.float32) + bo_ref[...]
    hh = h_ref[...] + hh
    mu = jnp.mean(hh, axis=0)
    var = jnp.mean((hh - mu) ** 2, axis=0)
    hh = (hh - mu) * lax.rsqrt(var + 1e-5) * g1_ref[...] + be1_ref[...]
    f = lax.dot_general(hh, w1_ref[...], dn,
                        preferred_element_type=jnp.float32) + b1_ref[...]
    f = jnp.maximum(f, 0.0)
    f = lax.dot_general(f, w2_ref[...], dn,
                        preferred_element_type=jnp.float32) + b2_ref[...]
    hh = hh + f
    mu2 = jnp.mean(hh, axis=0)
    var2 = jnp.mean((hh - mu2) ** 2, axis=0)
    out_ref[...] = ((hh - mu2) * lax.rsqrt(var2 + 1e-5) * g2_ref[...]
                    + be2_ref[...])


_S = np.repeat(np.eye(H, dtype=np.float32), DH, axis=1)


def kernel(h, edge_index, pos_enc, WQ, WK, WV, WO, bO, W1, b1, W2, b2,
           g1, be1, g2, be2):
    ei = edge_index.astype(jnp.int32)
    pad = jnp.concatenate(
        [jnp.zeros((1, E_PAD - E), jnp.int32),
         jnp.full((1, E_PAD - E), NP - 1, jnp.int32)], axis=0)
    ei = jnp.concatenate([ei, pad], axis=1)

    q, k, v = pl.pallas_call(
        _qkv_body,
        out_shape=[jax.ShapeDtypeStruct((N, D), jnp.bfloat16),
                   jax.ShapeDtypeStruct((N, D), jnp.bfloat16),
                   jax.ShapeDtypeStruct((N, D), jnp.float32)],
    )(h, WQ, WK, WV)
    qp = lax.bitcast_convert_type(q.reshape(N, D // 2, 2), jnp.int32)
    kp = lax.bitcast_convert_type(k.reshape(N, D // 2, 2), jnp.int32)
    qp = jnp.concatenate([qp, jnp.zeros((NP - N, D // 2), jnp.int32)], axis=0)

    wvp, zp = _edge_kernel(qp, kp, v, ei)

    out = pl.pallas_call(
        _post_body,
        out_shape=jax.ShapeDtypeStruct((N, D), jnp.float32),
    )(h, wvp, zp, jnp.asarray(_S), WO, bO, W1, b1, W2, b2, g1, be1, g2, be2)
    return out


# speedup vs baseline: 1.6033x; 1.3959x over previous
import functools

import jax
import jax.numpy as jnp
import numpy as np
from jax import lax
from jax.experimental import pallas as pl
from jax.experimental.pallas import tpu as pltpu
from jax.experimental.pallas import tpu_sc as plsc

N = 10000
E = 320000
D = 128
H = 8
DH = 16

NC = 2
NS = 16
NW = NC * NS
C = 80
NCHUNK = 126
EPT = C * NCHUNK
E_PAD = NW * EPT
G = C // 16
NP = 10112
RPT = NP // NS


def _qkv_body(h_ref, wq_ref, wk_ref, wv_ref, q_out, k_out, v_out):
    x = h_ref[...]
    dn = (((1,), (1,)), ((), ()))
    q_out[...] = lax.dot_general(x, wq_ref[...], dn,
                                 preferred_element_type=jnp.float32)
    k_out[...] = lax.dot_general(x, wk_ref[...], dn,
                                 preferred_element_type=jnp.float32)
    v_out[...] = lax.dot_general(x, wv_ref[...], dn,
                                 preferred_element_type=jnp.float32
                                 ).astype(jnp.bfloat16)


def _edge_body(q_hbm, k_hbm, v_hbm, ei_hbm,
               wv_out, z_out,
               k0, q0, k1, q1, v_buf, i0, i1, z_o, z_s,
               wv_acc, z_acc,
               semk0, semq0, semk1, semq1, semv, semw, semz):
    cid = lax.axis_index("c")
    sid = lax.axis_index("s")
    wid = sid * NC + cid
    lv = lax.iota(jnp.int32, 16)

    def zrow(r, _):
        for hh in range(4):
            v_buf[r, pl.ds(hh * 32, 32)] = jnp.zeros((32,), jnp.bfloat16)
        z_o[r, pl.ds(0, 16)] = jnp.zeros((16,), jnp.float32)
        return 0
    lax.fori_loop(0, C, zrow, 0)
    for j in range(RPT // C):
        base = sid * RPT + j * C
        pltpu.sync_copy(v_buf, wv_acc.at[pl.ds(base, C)])
        pltpu.sync_copy(z_o, z_acc.at[pl.ds(base, C)])
    rem = RPT - (RPT // C) * C
    base = sid * RPT + (RPT // C) * C
    pltpu.sync_copy(v_buf.at[pl.ds(0, rem)], wv_acc.at[pl.ds(base, rem)])
    pltpu.sync_copy(z_o.at[pl.ds(0, rem)], z_acc.at[pl.ds(base, rem)])
    plsc.subcore_barrier()

    def issue(ci, kb, qb, idx, semk, semq):
        base = wid * EPT + ci * C
        pltpu.sync_copy(ei_hbm.at[:, pl.ds(base, C)], idx)
        pltpu.async_copy(k_hbm.at[idx.at[0]], kb, semk)
        pltpu.async_copy(q_hbm.at[idx.at[1]], qb, semq)

    def process(ci, kb, qb, idx, semk, semq):
        src_i = idx.at[0]
        dst_i = idx.at[1]
        cv = pltpu.async_copy(v_hbm.at[src_i], v_buf, semv)
        pltpu.make_async_copy(k_hbm.at[src_i], kb, semk).wait()
        pltpu.make_async_copy(q_hbm.at[dst_i], qb, semq).wait()

        def score_body(g, _):
            ev = g * 16 + lv
            for h in range(H):
                acc0 = jnp.zeros((16,), jnp.float32)
                acc1 = jnp.zeros((16,), jnp.float32)
                for i in range(DH):
                    cvec = h * 16 + ((lv + i) & 15)
                    kv = plsc.load_gather(kb, [ev, cvec])
                    qv = plsc.load_gather(qb, [ev, cvec])
                    if i % 2 == 0:
                        acc0 = acc0 + kv * qv
                    else:
                        acc1 = acc1 + kv * qv
                sh = jnp.exp(jnp.clip((acc0 + acc1) * 0.25, -5.0, 5.0))
                plsc.store_scatter(z_s, [ev, jnp.full((16,), h, jnp.int32)], sh)
            return 0
        lax.fori_loop(0, G, score_body, 0)

        def zcopy_body(r, _):
            row = plsc.load_gather(z_s, [jnp.full((16,), r, jnp.int32), lv])
            z_o[r, pl.ds(0, 16)] = row
            return 0
        lax.fori_loop(0, C, zcopy_body, 0)

        cv.wait()

        def wv_body(e, _):
            rows = jnp.full((16,), 1, jnp.int32) * e
            for j in range(4):
                vv = v_buf[e, pl.ds(32 * j, 32)]
                va, vb = plsc.unpack(vv, format=plsc.PackFormat.INTERLEAVED)
                m = plsc.load_gather(z_o, [rows, 2 * j + (lv >> 3)])
                pk = plsc.pack(va * m, vb * m,
                               format=plsc.PackFormat.INTERLEAVED)
                v_buf[e, pl.ds(32 * j, 32)] = pk
            return 0
        lax.fori_loop(0, C, wv_body, 0)

        sa = pltpu.async_copy(v_buf, wv_acc.at[dst_i], semw, add=True)
        sz = pltpu.async_copy(z_o, z_acc.at[dst_i], semz, add=True)
        sa.wait()
        sz.wait()

    issue(0, k0, q0, i0, semk0, semq0)

    def pair_body(p, _):
        c0 = 2 * p
        issue(c0 + 1, k1, q1, i1, semk1, semq1)
        process(c0, k0, q0, i0, semk0, semq0)
        issue(c0 + 2, k0, q0, i0, semk0, semq0)
        process(c0 + 1, k1, q1, i1, semk1, semq1)
        return 0
    lax.fori_loop(0, NCHUNK // 2 - 1, pair_body, 0)

    issue(NCHUNK - 1, k1, q1, i1, semk1, semq1)
    process(NCHUNK - 2, k0, q0, i0, semk0, semq0)
    process(NCHUNK - 1, k1, q1, i1, semk1, semq1)

    plsc.subcore_barrier()

    for j in range(RPT // C):
        base = sid * RPT + j * C
        pltpu.sync_copy(wv_acc.at[pl.ds(base, C)],
                        wv_out.at[cid, pl.ds(base, C)])
        pltpu.sync_copy(z_acc.at[pl.ds(base, C)],
                        z_out.at[cid, pl.ds(base, C)])
    base = sid * RPT + (RPT // C) * C
    pltpu.sync_copy(wv_acc.at[pl.ds(base, rem)],
                    wv_out.at[cid, pl.ds(base, rem)])
    pltpu.sync_copy(z_acc.at[pl.ds(base, rem)],
                    z_out.at[cid, pl.ds(base, rem)])


_edge_kernel = functools.partial(
    pl.kernel,
    out_type=[jax.ShapeDtypeStruct((NC, NP, D), jnp.bfloat16),
              jax.ShapeDtypeStruct((NC, NP, 16), jnp.float32)],
    mesh=plsc.VectorSubcoreMesh(core_axis_name="c", subcore_axis_name="s"),
    compiler_params=pltpu.CompilerParams(needs_layout_passes=False,
                                         use_tc_tiling_on_sc=False),
    scratch_types=[
        pltpu.VMEM((C, D), jnp.float32),
        pltpu.VMEM((C, D), jnp.float32),
        pltpu.VMEM((C, D), jnp.float32),
        pltpu.VMEM((C, D), jnp.float32),
        pltpu.VMEM((C, D), jnp.bfloat16),
        pltpu.VMEM((2, C), jnp.int32),
        pltpu.VMEM((2, C), jnp.int32),
        pltpu.VMEM((C, 16), jnp.float32),
        pltpu.VMEM((C, 17), jnp.float32),
        pltpu.VMEM_SHARED((NP, D), jnp.bfloat16),
        pltpu.VMEM_SHARED((NP, 16), jnp.float32),
        pltpu.SemaphoreType.DMA,
        pltpu.SemaphoreType.DMA,
        pltpu.SemaphoreType.DMA,
        pltpu.SemaphoreType.DMA,
        pltpu.SemaphoreType.DMA,
        pltpu.SemaphoreType.DMA,
        pltpu.SemaphoreType.DMA,
    ],
)(_edge_body)


def _post_body(h_ref, wvp_ref, zp_ref, s_ref, wo_ref, bo_ref,
               w1_ref, b1_ref, w2_ref, b2_ref,
               g1_ref, be1_ref, g2_ref, be2_ref, out_ref):
    wv = (wvp_ref[0, 0:N].astype(jnp.float32)
          + wvp_ref[1, 0:N].astype(jnp.float32))
    z = zp_ref[0, 0:N, 0:8] + zp_ref[1, 0:N, 0:8]
    dn = (((1,), (1,)), ((), ()))
    dn0 = (((1,), (0,)), ((), ()))
    zx = lax.dot_general(1.0 / z, s_ref[...], dn0,
                         preferred_element_type=jnp.float32)
    head = wv * zx
    hh = lax.dot_general(head, wo_ref[...], dn,
                         preferred_element_type=jnp.float32) + bo_ref[...]
    hh = h_ref[...] + hh
    mu = jnp.mean(hh, axis=0)
    var = jnp.mean((hh - mu) ** 2, axis=0)
    hh = (hh - mu) * lax.rsqrt(var + 1e-5) * g1_ref[...] + be1_ref[...]
    f = lax.dot_general(hh, w1_ref[...], dn,
                        preferred_element_type=jnp.float32) + b1_ref[...]
    f = jnp.maximum(f, 0.0)
    f = lax.dot_general(f, w2_ref[...], dn,
                        preferred_element_type=jnp.float32) + b2_ref[...]
    hh = hh + f
    mu2 = jnp.mean(hh, axis=0)
    var2 = jnp.mean((hh - mu2) ** 2, axis=0)
    out_ref[...] = ((hh - mu2) * lax.rsqrt(var2 + 1e-5) * g2_ref[...]
                    + be2_ref[...])


_S = np.repeat(np.eye(H, dtype=np.float32), DH, axis=1)


def kernel(h, edge_index, pos_enc, WQ, WK, WV, WO, bO, W1, b1, W2, b2,
           g1, be1, g2, be2):
    ei = edge_index.astype(jnp.int32)
    pad = jnp.concatenate(
        [jnp.zeros((1, E_PAD - E), jnp.int32),
         jnp.full((1, E_PAD - E), NP - 1, jnp.int32)], axis=0)
    ei = jnp.concatenate([ei, pad], axis=1)

    q, k, v = pl.pallas_call(
        _qkv_body,
        out_shape=[jax.ShapeDtypeStruct((N, D), jnp.float32),
                   jax.ShapeDtypeStruct((N, D), jnp.float32),
                   jax.ShapeDtypeStruct((N, D), jnp.bfloat16)],
    )(h, WQ, WK, WV)
    q = jnp.concatenate([q, jnp.zeros((NP - N, D), jnp.float32)], axis=0)

    wvp, zp = _edge_kernel(q, k, v, ei)

    out = pl.pallas_call(
        _post_body,
        out_shape=jax.ShapeDtypeStruct((N, D), jnp.float32),
    )(h, wvp, zp, jnp.asarray(_S), WO, bO, W1, b1, W2, b2, g1, be1, g2, be2)
    return out


# speedup vs baseline: 1.6733x; 1.0437x over previous
import functools

import jax
import jax.numpy as jnp
import numpy as np
from jax import lax
from jax.experimental import pallas as pl
from jax.experimental.pallas import tpu as pltpu
from jax.experimental.pallas import tpu_sc as plsc

N = 10000
E = 320000
D = 128
H = 8
DH = 16

NC = 2
NS = 16
NW = NC * NS
C = 112
NCHUNK = 90
EPT = C * NCHUNK
E_PAD = NW * EPT
G = C // 16
NP = 10112
RPT = NP // NS


def _qkv_body(h_ref, wq_ref, wk_ref, wv_ref, q_out, k_out, v_out):
    x = h_ref[...]
    dn = (((1,), (1,)), ((), ()))
    q_out[...] = lax.dot_general(x, wq_ref[...], dn,
                                 preferred_element_type=jnp.float32)
    k_out[...] = lax.dot_general(x, wk_ref[...], dn,
                                 preferred_element_type=jnp.float32)
    v_out[...] = lax.dot_general(x, wv_ref[...], dn,
                                 preferred_element_type=jnp.float32
                                 ).astype(jnp.bfloat16)


def _edge_body(q_hbm, k_hbm, v_hbm, ei_hbm,
               wv_out, z_out,
               k0, q0, k1, q1, v_buf, i0, i1, z_o, z_s,
               wv_acc, z_acc,
               semk0, semq0, semk1, semq1, semv, semw, semz):
    cid = lax.axis_index("c")
    sid = lax.axis_index("s")
    wid = sid * NC + cid
    lv = lax.iota(jnp.int32, 16)

    def zrow(r, _):
        for hh in range(4):
            v_buf[r, pl.ds(hh * 32, 32)] = jnp.zeros((32,), jnp.bfloat16)
        return 0
    lax.fori_loop(0, C, zrow, 0)

    def zzrow(t, _):
        rows = 2 * t + (lv >> 3)
        plsc.store_scatter(z_o, [rows, lv & 7], jnp.zeros((16,), jnp.float32))
        return 0
    lax.fori_loop(0, C // 2, zzrow, 0)
    for j in range(RPT // C):
        base = sid * RPT + j * C
        pltpu.sync_copy(v_buf, wv_acc.at[pl.ds(base, C)])
        pltpu.sync_copy(z_o, z_acc.at[pl.ds(base, C)])
    rem = RPT - (RPT // C) * C
    base = sid * RPT + (RPT // C) * C
    pltpu.sync_copy(v_buf.at[pl.ds(0, rem)], wv_acc.at[pl.ds(base, rem)])
    pltpu.sync_copy(z_o.at[pl.ds(0, rem)], z_acc.at[pl.ds(base, rem)])
    plsc.subcore_barrier()

    def issue(ci, kb, qb, idx, semk, semq):
        base = wid * EPT + ci * C
        pltpu.sync_copy(ei_hbm.at[:, pl.ds(base, C)], idx)
        pltpu.async_copy(k_hbm.at[idx.at[0]], kb, semk)
        pltpu.async_copy(q_hbm.at[idx.at[1]], qb, semq)

    def process(ci, kb, qb, idx, semk, semq):
        src_i = idx.at[0]
        dst_i = idx.at[1]
        cv = pltpu.async_copy(v_hbm.at[src_i], v_buf, semv)
        pltpu.make_async_copy(k_hbm.at[src_i], kb, semk).wait()
        pltpu.make_async_copy(q_hbm.at[dst_i], qb, semq).wait()

        def score_body(g, _):
            ev = g * 16 + lv
            for h in range(H):
                acc0 = jnp.zeros((16,), jnp.float32)
                acc1 = jnp.zeros((16,), jnp.float32)
                for i in range(DH):
                    cvec = h * 16 + ((lv + i) & 15)
                    kv = plsc.load_gather(kb, [ev, cvec])
                    qv = plsc.load_gather(qb, [ev, cvec])
                    if i % 2 == 0:
                        acc0 = acc0 + kv * qv
                    else:
                        acc1 = acc1 + kv * qv
                sh = jnp.exp(jnp.clip((acc0 + acc1) * 0.25, -5.0, 5.0))
                plsc.store_scatter(z_s, [ev, jnp.full((16,), h, jnp.int32)], sh)
            return 0
        lax.fori_loop(0, G, score_body, 0)

        def zcopy_body(t, _):
            rows = 2 * t + (lv >> 3)
            cols = lv & 7
            val = plsc.load_gather(z_s, [rows, cols])
            plsc.store_scatter(z_o, [rows, cols], val)
            return 0
        lax.fori_loop(0, C // 2, zcopy_body, 0)

        cv.wait()

        def wv_body(e, _):
            rows = jnp.full((16,), 1, jnp.int32) * e
            for j in range(4):
                vv = v_buf[e, pl.ds(32 * j, 32)]
                va, vb = plsc.unpack(vv, format=plsc.PackFormat.INTERLEAVED)
                m = plsc.load_gather(z_o, [rows, 2 * j + (lv >> 3)])
                pk = plsc.pack(va * m, vb * m,
                               format=plsc.PackFormat.INTERLEAVED)
                v_buf[e, pl.ds(32 * j, 32)] = pk
            return 0
        lax.fori_loop(0, C, wv_body, 0)

        sa = pltpu.async_copy(v_buf, wv_acc.at[dst_i], semw, add=True)
        sz = pltpu.async_copy(z_o, z_acc.at[dst_i], semz, add=True)
        sa.wait()
        sz.wait()

    issue(0, k0, q0, i0, semk0, semq0)

    def pair_body(p, _):
        c0 = 2 * p
        issue(c0 + 1, k1, q1, i1, semk1, semq1)
        process(c0, k0, q0, i0, semk0, semq0)
        issue(c0 + 2, k0, q0, i0, semk0, semq0)
        process(c0 + 1, k1, q1, i1, semk1, semq1)
        return 0
    lax.fori_loop(0, NCHUNK // 2 - 1, pair_body, 0)

    issue(NCHUNK - 1, k1, q1, i1, semk1, semq1)
    process(NCHUNK - 2, k0, q0, i0, semk0, semq0)
    process(NCHUNK - 1, k1, q1, i1, semk1, semq1)

    plsc.subcore_barrier()

    for j in range(RPT // C):
        base = sid * RPT + j * C
        pltpu.sync_copy(wv_acc.at[pl.ds(base, C)],
                        wv_out.at[cid, pl.ds(base, C)])
        pltpu.sync_copy(z_acc.at[pl.ds(base, C)],
                        z_out.at[cid, pl.ds(base, C)])
    base = sid * RPT + (RPT // C) * C
    pltpu.sync_copy(wv_acc.at[pl.ds(base, rem)],
                    wv_out.at[cid, pl.ds(base, rem)])
    pltpu.sync_copy(z_acc.at[pl.ds(base, rem)],
                    z_out.at[cid, pl.ds(base, rem)])


_edge_kernel = functools.partial(
    pl.kernel,
    out_type=[jax.ShapeDtypeStruct((NC, NP, D), jnp.bfloat16),
              jax.ShapeDtypeStruct((NC, NP, 8), jnp.float32)],
    mesh=plsc.VectorSubcoreMesh(core_axis_name="c", subcore_axis_name="s"),
    compiler_params=pltpu.CompilerParams(needs_layout_passes=False,
                                         use_tc_tiling_on_sc=False),
    scratch_types=[
        pltpu.VMEM((C, D), jnp.float32),
        pltpu.VMEM((C, D), jnp.float32),
        pltpu.VMEM((C, D), jnp.float32),
        pltpu.VMEM((C, D), jnp.float32),
        pltpu.VMEM((C, D), jnp.bfloat16),
        pltpu.VMEM((2, C), jnp.int32),
        pltpu.VMEM((2, C), jnp.int32),
        pltpu.VMEM((C, 8), jnp.float32),
        pltpu.VMEM((C, 17), jnp.float32),
        pltpu.VMEM_SHARED((NP, D), jnp.bfloat16),
        pltpu.VMEM_SHARED((NP, 8), jnp.float32),
        pltpu.SemaphoreType.DMA,
        pltpu.SemaphoreType.DMA,
        pltpu.SemaphoreType.DMA,
        pltpu.SemaphoreType.DMA,
        pltpu.SemaphoreType.DMA,
        pltpu.SemaphoreType.DMA,
        pltpu.SemaphoreType.DMA,
    ],
)(_edge_body)


def _post_body(h_ref, wvp_ref, zp_ref, s_ref, wo_ref, bo_ref,
               w1_ref, b1_ref, w2_ref, b2_ref,
               g1_ref, be1_ref, g2_ref, be2_ref, out_ref):
    wv = (wvp_ref[0, 0:N].astype(jnp.float32)
          + wvp_ref[1, 0:N].astype(jnp.float32))
    z = zp_ref[0, 0:N] + zp_ref[1, 0:N]
    dn = (((1,), (1,)), ((), ()))
    dn0 = (((1,), (0,)), ((), ()))
    zx = lax.dot_general(1.0 / z, s_ref[...], dn0,
                         preferred_element_type=jnp.float32)
    head = wv * zx
    hh = lax.dot_general(head, wo_ref[...], dn,
                         preferred_element_type=jnp.float32) + bo_ref[...]
    hh = h_ref[...] + hh
    mu = jnp.mean(hh, axis=0)
    var = jnp.mean((hh - mu) ** 2, axis=0)
    hh = (hh - mu) * lax.rsqrt(var + 1e-5) * g1_ref[...] + be1_ref[...]
    f = lax.dot_general(hh, w1_ref[...], dn,
                        preferred_element_type=jnp.float32) + b1_ref[...]
    f = jnp.maximum(f, 0.0)
    f = lax.dot_general(f, w2_ref[...], dn,
                        preferred_element_type=jnp.float32) + b2_ref[...]
    hh = hh + f
    mu2 = jnp.mean(hh, axis=0)
    var2 = jnp.mean((hh - mu2) ** 2, axis=0)
    out_ref[...] = ((hh - mu2) * lax.rsqrt(var2 + 1e-5) * g2_ref[...]
                    + be2_ref[...])


_S = np.repeat(np.eye(H, dtype=np.float32), DH, axis=1)


def kernel(h, edge_index, pos_enc, WQ, WK, WV, WO, bO, W1, b1, W2, b2,
           g1, be1, g2, be2):
    ei = edge_index.astype(jnp.int32)
    pad = jnp.concatenate(
        [jnp.zeros((1, E_PAD - E), jnp.int32),
         jnp.full((1, E_PAD - E), NP - 1, jnp.int32)], axis=0)
    ei = jnp.concatenate([ei, pad], axis=1)

    q, k, v = pl.pallas_call(
        _qkv_body,
        out_shape=[jax.ShapeDtypeStruct((N, D), jnp.float32),
                   jax.ShapeDtypeStruct((N, D), jnp.float32),
                   jax.ShapeDtypeStruct((N, D), jnp.bfloat16)],
    )(h, WQ, WK, WV)
    q = jnp.concatenate([q, jnp.zeros((NP - N, D), jnp.float32)], axis=0)

    wvp, zp = _edge_kernel(q, k, v, ei)

    out = pl.pallas_call(
        _post_body,
        out_shape=jax.ShapeDtypeStruct((N, D), jnp.float32),
    )(h, wvp, zp, jnp.asarray(_S), WO, bO, W1, b1, W2, b2, g1, be1, g2, be2)
    return out
